# Initial kernel scaffold; baseline (speedup 1.0000x reference)
import functools

import jax
import jax.numpy as jnp
from jax import lax
from jax.experimental import pallas as pl
from jax.experimental.pallas import tpu as pltpu
from jax.experimental.pallas import tpu_sc as plsc

LANES = 16
NSUB = 16
NCORES = 2
CHUNK = 128


@functools.lru_cache(maxsize=None)
def _make_agg(N, E, DH):
    assert E % CHUNK == 0 and N % NSUB == 0 and DH % LANES == 0
    n_chunks = E // CHUNK
    outer = (n_chunks + NSUB - 1) // NSUB
    rows_per_sub = N // NSUB
    mesh = plsc.VectorSubcoreMesh(core_axis_name="c", subcore_axis_name="s")

    @functools.partial(
        pl.kernel,
        out_type=jax.ShapeDtypeStruct((NCORES * N, DH), jnp.float32),
        mesh=mesh,
        scratch_types=[
            pltpu.VMEM((CHUNK,), jnp.int32),
            pltpu.VMEM((CHUNK,), jnp.int32),
            pltpu.VMEM((CHUNK,), jnp.int32),
            pltpu.VMEM((CHUNK,), jnp.int32),
            pltpu.VMEM((CHUNK, DH), jnp.float32),
            pltpu.VMEM((CHUNK, DH), jnp.float32),
            pltpu.VMEM_SHARED((N, DH), jnp.float32),
            pltpu.SemaphoreType.DMA,
            pltpu.SemaphoreType.DMA,
        ],
    )
    def agg_kernel(xh, src_h, dst_h, ea_h, zeros_h, out_h,
                   srcv, dstv, xidx, eidx, xbuf, eabuf, aggs, sem1, sem2):
        c = lax.axis_index("c")
        s = lax.axis_index("s")
        r0 = s * rows_per_sub
        pltpu.sync_copy(zeros_h, aggs.at[pl.ds(r0, rows_per_sub)])
        plsc.subcore_barrier()
        iota = lax.iota(jnp.int32, LANES)

        @pl.loop(0, outer)
        def _chunks(t):
            g = t * NSUB + s

            @pl.when(g < n_chunks)
            def _():
                base = g * CHUNK
                pltpu.sync_copy(src_h.at[pl.ds(base, CHUNK)], srcv)
                pltpu.sync_copy(dst_h.at[pl.ds(base, CHUNK)], dstv)
                for j in range(CHUNK // LANES):
                    sl = pl.ds(j * LANES, LANES)
                    xidx[sl] = srcv[sl] * 2 + c
                    eidx[sl] = (base + j * LANES + iota) * 2 + c
                cp1 = pltpu.async_copy(xh.at[xidx], xbuf, sem1)
                cp2 = pltpu.async_copy(ea_h.at[eidx], eabuf, sem2)
                cp1.wait()
                cp2.wait()

                @pl.loop(0, CHUNK)
                def _rows(r):
                    for j in range(DH // LANES):
                        sl = pl.ds(j * LANES, LANES)
                        eabuf[r, sl] = jnp.maximum(xbuf[r, sl] + eabuf[r, sl], 0.0)

                pltpu.sync_copy(eabuf, aggs.at[dstv], add=True)

        plsc.subcore_barrier()
        pltpu.sync_copy(aggs.at[pl.ds(r0, rows_per_sub)],
                        out_h.at[pl.ds(c * N + r0, rows_per_sub)])

    return agg_kernel


def _mlp_body(eps_ref, x_ref, a0_ref, a1_ref, w1_ref, b1_ref, w2_ref, b2_ref, o_ref):
    agg = jnp.concatenate([a0_ref[...], a1_ref[...]], axis=1)
    h = x_ref[...] * (1.0 + eps_ref[0, 0]) + agg
    h = jnp.maximum(jnp.dot(h, w1_ref[...], preferred_element_type=jnp.float32)
                    + b1_ref[...], 0.0)
    h = jnp.dot(h, w2_ref[...], preferred_element_type=jnp.float32) + b2_ref[...]
    o_ref[...] = jnp.maximum(h, 0.0)


@functools.lru_cache(maxsize=None)
def _make_mlp(N, D, BN):
    DH = D // 2
    nb = N // BN
    assert N % BN == 0
    return pl.pallas_call(
        _mlp_body,
        grid=(nb,),
        in_specs=[
            pl.BlockSpec(memory_space=pltpu.SMEM),
            pl.BlockSpec((BN, D), lambda i: (i, 0)),
            pl.BlockSpec((BN, DH), lambda i: (i, 0)),
            pl.BlockSpec((BN, DH), lambda i: (i + nb, 0)),
            pl.BlockSpec((D, D), lambda i: (0, 0)),
            pl.BlockSpec((1, D), lambda i: (0, 0)),
            pl.BlockSpec((D, D), lambda i: (0, 0)),
            pl.BlockSpec((1, D), lambda i: (0, 0)),
        ],
        out_specs=pl.BlockSpec((BN, D), lambda i: (i, 0)),
        out_shape=jax.ShapeDtypeStruct((N, D), jnp.float32),
    )


def kernel(x, edge_index, edge_attr, W1, b1, W2, b2, eps):
    N, D = x.shape
    E = edge_index.shape[1]
    DH = D // 2
    xh = x.reshape(N * 2, DH)
    eah = edge_attr.reshape(E * 2, DH)
    src = edge_index[0]
    dst = edge_index[1]
    zeros = jnp.zeros((N // NSUB, DH), jnp.float32)
    aggflat = _make_agg(N, E, DH)(xh, src, dst, eah, zeros)

    li = W1.shape[0] - 1
    out = _make_mlp(N, D, 1000)(
        eps[li].reshape(1, 1), x, aggflat, aggflat,
        W1[li], b1[li].reshape(1, D), W2[li], b2[li].reshape(1, D))
    return out


# speedup vs baseline: 2.2973x; 2.2973x over previous
import functools

import jax
import jax.numpy as jnp
from jax import lax
from jax.experimental import pallas as pl
from jax.experimental.pallas import tpu as pltpu
from jax.experimental.pallas import tpu_sc as plsc

LANES = 16
NSUB = 16
NCORES = 2
CHUNK = 128


@functools.lru_cache(maxsize=None)
def _make_agg(N, E, DH):
    assert E % CHUNK == 0 and N % 8 == 0 and DH % LANES == 0
    n_chunks = E // CHUNK
    outer = (n_chunks + NSUB - 1) // NSUB
    rows_a = (N // NSUB) // 8 * 8
    rows_last = N - (NSUB - 1) * rows_a
    mesh = plsc.VectorSubcoreMesh(core_axis_name="c", subcore_axis_name="s")

    @functools.partial(
        pl.kernel,
        out_type=jax.ShapeDtypeStruct((NCORES * N, DH), jnp.float32),
        mesh=mesh,
        scratch_types=[
            pltpu.VMEM((CHUNK,), jnp.int32),
            pltpu.VMEM((CHUNK,), jnp.int32),
            pltpu.VMEM((CHUNK,), jnp.int32),
            pltpu.VMEM((CHUNK,), jnp.int32),
            pltpu.VMEM((CHUNK, DH), jnp.float32),
            pltpu.VMEM((CHUNK, DH), jnp.float32),
            pltpu.VMEM_SHARED((N, DH), jnp.float32),
            pltpu.SemaphoreType.DMA,
            pltpu.SemaphoreType.DMA,
        ],
    )
    def agg_kernel(xh, src_h, dst_h, ea_h, zeros_h, out_h,
                   srcv, dstv, xidx, eidx, xbuf, eabuf, aggs, sem1, sem2):
        c = lax.axis_index("c")
        s = lax.axis_index("s")
        r0 = pl.multiple_of(s * rows_a, 8)
        o0 = pl.multiple_of(c * N + s * rows_a, 8)

        @pl.when(s < NSUB - 1)
        def _():
            pltpu.sync_copy(zeros_h.at[pl.ds(0, rows_a)],
                            aggs.at[pl.ds(r0, rows_a)])

        @pl.when(s == NSUB - 1)
        def _():
            pltpu.sync_copy(zeros_h, aggs.at[pl.ds((NSUB - 1) * rows_a, rows_last)])

        plsc.subcore_barrier()
        iota = lax.iota(jnp.int32, LANES)

        @pl.loop(0, outer)
        def _chunks(t):
            g = t * NSUB + s

            @pl.when(g < n_chunks)
            def _():
                base = g * CHUNK
                pltpu.sync_copy(src_h.at[pl.ds(base, CHUNK)], srcv)
                pltpu.sync_copy(dst_h.at[pl.ds(base, CHUNK)], dstv)
                for j in range(CHUNK // LANES):
                    sl = pl.ds(j * LANES, LANES)
                    xidx[sl] = srcv[sl] * 2 + c
                    eidx[sl] = (base + j * LANES + iota) * 2 + c
                cp1 = pltpu.async_copy(xh.at[xidx], xbuf, sem1)
                cp2 = pltpu.async_copy(ea_h.at[eidx], eabuf, sem2)
                cp1.wait()
                cp2.wait()

                @pl.loop(0, CHUNK)
                def _rows(r):
                    for j in range(DH // LANES):
                        sl = pl.ds(j * LANES, LANES)
                        eabuf[r, sl] = jnp.maximum(xbuf[r, sl] + eabuf[r, sl], 0.0)

                pltpu.sync_copy(eabuf, aggs.at[dstv], add=True)

        plsc.subcore_barrier()

        @pl.when(s < NSUB - 1)
        def _():
            pltpu.sync_copy(aggs.at[pl.ds(r0, rows_a)],
                            out_h.at[pl.ds(o0, rows_a)])

        @pl.when(s == NSUB - 1)
        def _():
            pltpu.sync_copy(
                aggs.at[pl.ds((NSUB - 1) * rows_a, rows_last)],
                out_h.at[pl.ds(pl.multiple_of(c * N, 8) + (NSUB - 1) * rows_a,
                               rows_last)])

    return agg_kernel


def _mlp_body(eps_ref, x_ref, a0_ref, a1_ref, w1_ref, b1_ref, w2_ref, b2_ref, o_ref):
    agg = jnp.concatenate([a0_ref[...], a1_ref[...]], axis=1)
    h = x_ref[...] * (1.0 + eps_ref[0, 0]) + agg
    h = jnp.maximum(jnp.dot(h, w1_ref[...], preferred_element_type=jnp.float32)
                    + b1_ref[...], 0.0)
    h = jnp.dot(h, w2_ref[...], preferred_element_type=jnp.float32) + b2_ref[...]
    o_ref[...] = jnp.maximum(h, 0.0)


@functools.lru_cache(maxsize=None)
def _make_mlp(N, D, BN):
    DH = D // 2
    nb = N // BN
    assert N % BN == 0
    return pl.pallas_call(
        _mlp_body,
        grid=(nb,),
        in_specs=[
            pl.BlockSpec(memory_space=pltpu.SMEM),
            pl.BlockSpec((BN, D), lambda i: (i, 0)),
            pl.BlockSpec((BN, DH), lambda i: (i, 0)),
            pl.BlockSpec((BN, DH), lambda i: (i + nb, 0)),
            pl.BlockSpec((D, D), lambda i: (0, 0)),
            pl.BlockSpec((1, D), lambda i: (0, 0)),
            pl.BlockSpec((D, D), lambda i: (0, 0)),
            pl.BlockSpec((1, D), lambda i: (0, 0)),
        ],
        out_specs=pl.BlockSpec((BN, D), lambda i: (i, 0)),
        out_shape=jax.ShapeDtypeStruct((N, D), jnp.float32),
    )


def kernel(x, edge_index, edge_attr, W1, b1, W2, b2, eps):
    N, D = x.shape
    E = edge_index.shape[1]
    DH = D // 2
    xh = x.reshape(N * 2, DH)
    eah = edge_attr.reshape(E * 2, DH)
    src = edge_index[0]
    dst = edge_index[1]
    rows_last = N - (NSUB - 1) * ((N // NSUB) // 8 * 8)
    zeros = jnp.zeros((rows_last, DH), jnp.float32)
    aggflat = _make_agg(N, E, DH)(xh, src, dst, eah, zeros)

    li = W1.shape[0] - 1
    out = _make_mlp(N, D, 1000)(
        eps[li].reshape(1, 1), x, aggflat, aggflat,
        W1[li], b1[li].reshape(1, D), W2[li], b2[li].reshape(1, D))
    return out


# speedup vs baseline: 3.4230x; 1.4900x over previous
import functools

import jax
import jax.numpy as jnp
from jax import lax
from jax.experimental import pallas as pl
from jax.experimental.pallas import tpu as pltpu
from jax.experimental.pallas import tpu_sc as plsc

LANES = 16
NSUB = 16
NCORES = 2
CHUNK = 80


@functools.lru_cache(maxsize=None)
def _make_agg(N, E, DH):
    assert N % 8 == 0 and DH % LANES == 0 and CHUNK % LANES == 0
    edges_per_sub = E // NSUB
    assert E % NSUB == 0 and edges_per_sub % CHUNK == 0
    n_chunks = edges_per_sub // CHUNK
    rows_a = (N // NSUB) // 8 * 8
    rows_last = N - (NSUB - 1) * rows_a
    mesh = plsc.VectorSubcoreMesh(core_axis_name="c", subcore_axis_name="s")

    @functools.partial(
        pl.kernel,
        out_type=jax.ShapeDtypeStruct((NCORES * N, DH), jnp.float32),
        mesh=mesh,
        scratch_types=[
            pltpu.VMEM((2, CHUNK), jnp.int32),
            pltpu.VMEM((4, CHUNK), jnp.int32),
            pltpu.VMEM((2, CHUNK), jnp.int32),
            pltpu.VMEM((2, CHUNK), jnp.int32),
            pltpu.VMEM((2, CHUNK, DH), jnp.float32),
            pltpu.VMEM((2, CHUNK, DH), jnp.float32),
            pltpu.VMEM_SHARED((N, DH), jnp.float32),
            [pltpu.SemaphoreType.DMA] * 2,
            [pltpu.SemaphoreType.DMA] * 2,
            [pltpu.SemaphoreType.DMA] * 2,
            [pltpu.SemaphoreType.DMA] * 2,
            [pltpu.SemaphoreType.DMA] * 2,
        ],
    )
    def agg_kernel(xh, src_h, dst_h, ea_h, zeros_h, out_h,
                   srcr, dstr, xidx, eidx, xbuf, eabuf, aggs,
                   semsrc, semdst, semx, seme, sems):
        c = lax.axis_index("c")
        s = lax.axis_index("s")
        r0 = pl.multiple_of(s * rows_a, 8)
        o0 = pl.multiple_of(c * N + s * rows_a, 8)
        iota = lax.iota(jnp.int32, LANES)
        e_base = pl.multiple_of(s * edges_per_sub, 8)

        @pl.when(s < NSUB - 1)
        def _():
            pltpu.sync_copy(zeros_h.at[pl.ds(0, rows_a)],
                            aggs.at[pl.ds(r0, rows_a)])

        @pl.when(s == NSUB - 1)
        def _():
            pltpu.sync_copy(zeros_h, aggs.at[pl.ds((NSUB - 1) * rows_a, rows_last)])

        plsc.subcore_barrier()

        def fire_idx_loads(t, s2, d4):
            pltpu.async_copy(src_h.at[pl.ds(e_base + t * CHUNK, CHUNK)],
                             srcr.at[s2], semsrc[s2])
            pltpu.async_copy(dst_h.at[pl.ds(e_base + t * CHUNK, CHUNK)],
                             dstr.at[d4], semdst[s2])

        def wait_idx_loads(t, s2, d4):
            pltpu.make_async_copy(src_h.at[pl.ds(e_base + t * CHUNK, CHUNK)],
                                  srcr.at[s2], semsrc[s2]).wait()
            pltpu.make_async_copy(dst_h.at[pl.ds(e_base + t * CHUNK, CHUNK)],
                                  dstr.at[d4], semdst[s2]).wait()

        def build_and_fire(t, b2):
            for j in range(CHUNK // LANES):
                sl = pl.ds(j * LANES, LANES)
                xidx[b2, sl] = srcr[b2, sl] * 2 + c
                eidx[b2, sl] = (e_base + t * CHUNK + j * LANES + iota) * 2 + c
            pltpu.async_copy(xh.at[xidx.at[b2]], xbuf.at[b2], semx[b2])
            pltpu.async_copy(ea_h.at[eidx.at[b2]], eabuf.at[b2], seme[b2])

        def wait_gathers(b2):
            pltpu.make_async_copy(xh.at[xidx.at[b2]], xbuf.at[b2], semx[b2]).wait()
            pltpu.make_async_copy(ea_h.at[eidx.at[b2]], eabuf.at[b2], seme[b2]).wait()

        def wait_scatter(b2, d4):
            pltpu.make_async_copy(eabuf.at[b2], aggs.at[dstr.at[d4]],
                                  sems[b2]).wait()

        fire_idx_loads(0, 0, 0)
        fire_idx_loads(1, 1, 1)
        wait_idx_loads(0, 0, 0)
        build_and_fire(0, 0)

        @pl.loop(0, n_chunks + 3, step=4)
        def _quad(t0):
            for qq in range(4):
                t = t0 + qq
                b2 = qq % 2
                d4 = qq

                @pl.when(t + 2 < n_chunks)
                def _():
                    fire_idx_loads(t + 2, b2, (d4 + 2) % 4)

                @pl.when((t >= 1) & (t < n_chunks))
                def _():
                    wait_scatter(1 - b2, (d4 + 3) % 4)

                @pl.when(t + 1 < n_chunks)
                def _():
                    wait_idx_loads(t + 1, 1 - b2, (d4 + 1) % 4)
                    build_and_fire(t + 1, 1 - b2)

                @pl.when(t < n_chunks)
                def _():
                    wait_gathers(b2)

                    @pl.loop(0, CHUNK)
                    def _rows(r):
                        for j in range(DH // LANES):
                            sl = pl.ds(j * LANES, LANES)
                            eabuf[b2, r, sl] = jnp.maximum(
                                xbuf[b2, r, sl] + eabuf[b2, r, sl], 0.0)

                    pltpu.async_copy(eabuf.at[b2], aggs.at[dstr.at[d4]],
                                     sems[b2], add=True)

        wait_scatter((n_chunks - 1) % 2, (n_chunks - 1) % 4)
        plsc.subcore_barrier()

        @pl.when(s < NSUB - 1)
        def _():
            pltpu.sync_copy(aggs.at[pl.ds(r0, rows_a)],
                            out_h.at[pl.ds(o0, rows_a)])

        @pl.when(s == NSUB - 1)
        def _():
            pltpu.sync_copy(
                aggs.at[pl.ds((NSUB - 1) * rows_a, rows_last)],
                out_h.at[pl.ds(pl.multiple_of(c * N, 8) + (NSUB - 1) * rows_a,
                               rows_last)])

    return agg_kernel


def _mlp_body(eps_ref, x_ref, a0_ref, a1_ref, w1_ref, b1_ref, w2_ref, b2_ref, o_ref):
    agg = jnp.concatenate([a0_ref[...], a1_ref[...]], axis=1)
    h = x_ref[...] * (1.0 + eps_ref[0, 0]) + agg
    h = jnp.maximum(jnp.dot(h, w1_ref[...], preferred_element_type=jnp.float32)
                    + b1_ref[...], 0.0)
    h = jnp.dot(h, w2_ref[...], preferred_element_type=jnp.float32) + b2_ref[...]
    o_ref[...] = jnp.maximum(h, 0.0)


@functools.lru_cache(maxsize=None)
def _make_mlp(N, D, BN):
    DH = D // 2
    nb = N // BN
    assert N % BN == 0
    return pl.pallas_call(
        _mlp_body,
        grid=(nb,),
        in_specs=[
            pl.BlockSpec(memory_space=pltpu.SMEM),
            pl.BlockSpec((BN, D), lambda i: (i, 0)),
            pl.BlockSpec((BN, DH), lambda i: (i, 0)),
            pl.BlockSpec((BN, DH), lambda i: (i + nb, 0)),
            pl.BlockSpec((D, D), lambda i: (0, 0)),
            pl.BlockSpec((1, D), lambda i: (0, 0)),
            pl.BlockSpec((D, D), lambda i: (0, 0)),
            pl.BlockSpec((1, D), lambda i: (0, 0)),
        ],
        out_specs=pl.BlockSpec((BN, D), lambda i: (i, 0)),
        out_shape=jax.ShapeDtypeStruct((N, D), jnp.float32),
    )


def kernel(x, edge_index, edge_attr, W1, b1, W2, b2, eps):
    N, D = x.shape
    E = edge_index.shape[1]
    DH = D // 2
    xh = x.reshape(N * 2, DH)
    eah = edge_attr.reshape(E * 2, DH)
    src = edge_index[0]
    dst = edge_index[1]
    rows_last = N - (NSUB - 1) * ((N // NSUB) // 8 * 8)
    zeros = jnp.zeros((rows_last, DH), jnp.float32)
    aggflat = _make_agg(N, E, DH)(xh, src, dst, eah, zeros)

    li = W1.shape[0] - 1
    out = _make_mlp(N, D, 1000)(
        eps[li].reshape(1, 1), x, aggflat, aggflat,
        W1[li], b1[li].reshape(1, D), W2[li], b2[li].reshape(1, D))
    return out


# speedup vs baseline: 5.6389x; 1.6473x over previous
import functools

import jax
import jax.numpy as jnp
from jax import lax
from jax.experimental import pallas as pl
from jax.experimental.pallas import tpu as pltpu
from jax.experimental.pallas import tpu_sc as plsc

LANES = 16
NSUB = 16
NCORES = 2
CHUNK = 80


@functools.lru_cache(maxsize=None)
def _make_agg(N, E, DH):
    assert N % 8 == 0 and DH % LANES == 0 and CHUNK % LANES == 0
    edges_per_sub = E // NSUB
    assert E % NSUB == 0 and edges_per_sub % CHUNK == 0
    n_chunks = edges_per_sub // CHUNK
    rows_a = (N // NSUB) // 8 * 8
    rows_last = N - (NSUB - 1) * rows_a
    mesh = plsc.VectorSubcoreMesh(core_axis_name="c", subcore_axis_name="s")

    @functools.partial(
        pl.kernel,
        out_type=jax.ShapeDtypeStruct((NCORES * N, DH), jnp.float32),
        mesh=mesh,
        scratch_types=[
            pltpu.VMEM((2, CHUNK), jnp.int32),
            pltpu.VMEM((4, CHUNK), jnp.int32),
            pltpu.VMEM((2, CHUNK), jnp.int32),
            pltpu.VMEM((2, CHUNK, DH), jnp.float32),
            pltpu.VMEM((2, CHUNK, DH), jnp.float32),
            pltpu.VMEM_SHARED((N, DH), jnp.float32),
            [pltpu.SemaphoreType.DMA] * 2,
            [pltpu.SemaphoreType.DMA] * 2,
            [pltpu.SemaphoreType.DMA] * 2,
            [pltpu.SemaphoreType.DMA] * 2,
            [pltpu.SemaphoreType.DMA] * 2,
        ],
    )
    def agg_kernel(xh, src_h, dst_h, ea_h, zeros_h, out_h,
                   srcr, dstr, xidx, xbuf, eabuf, aggs,
                   semsrc, semdst, semx, seme, sems):
        c = lax.axis_index("c")
        s = lax.axis_index("s")
        r0 = pl.multiple_of(s * rows_a, 8)
        o0 = pl.multiple_of(c * N + s * rows_a, 8)
        e_base = pl.multiple_of(s * edges_per_sub, 8)

        @pl.when(s < NSUB - 1)
        def _():
            pltpu.sync_copy(zeros_h.at[pl.ds(0, rows_a)],
                            aggs.at[pl.ds(r0, rows_a)])

        @pl.when(s == NSUB - 1)
        def _():
            pltpu.sync_copy(zeros_h, aggs.at[pl.ds((NSUB - 1) * rows_a, rows_last)])

        plsc.subcore_barrier()

        def fire_idx_loads(t, s2, d4):
            pltpu.async_copy(src_h.at[pl.ds(e_base + t * CHUNK, CHUNK)],
                             srcr.at[s2], semsrc[s2])
            pltpu.async_copy(dst_h.at[pl.ds(e_base + t * CHUNK, CHUNK)],
                             dstr.at[d4], semdst[s2])

        def wait_idx_loads(t, s2, d4):
            pltpu.make_async_copy(src_h.at[pl.ds(e_base + t * CHUNK, CHUNK)],
                                  srcr.at[s2], semsrc[s2]).wait()
            pltpu.make_async_copy(dst_h.at[pl.ds(e_base + t * CHUNK, CHUNK)],
                                  dstr.at[d4], semdst[s2]).wait()

        col0 = pl.multiple_of(c * DH, DH)

        def build_and_fire(t, b2):
            for j in range(CHUNK // LANES):
                sl = pl.ds(j * LANES, LANES)
                xidx[b2, sl] = srcr[b2, sl] * 2 + c
            pltpu.async_copy(xh.at[xidx.at[b2]], xbuf.at[b2], semx[b2])
            pltpu.async_copy(
                ea_h.at[pl.ds(e_base + t * CHUNK, CHUNK), pl.ds(col0, DH)],
                eabuf.at[b2], seme[b2])

        def wait_gathers(t, b2):
            pltpu.make_async_copy(xh.at[xidx.at[b2]], xbuf.at[b2], semx[b2]).wait()
            pltpu.make_async_copy(
                ea_h.at[pl.ds(e_base + t * CHUNK, CHUNK), pl.ds(col0, DH)],
                eabuf.at[b2], seme[b2]).wait()

        def wait_scatter(b2, d4):
            pltpu.make_async_copy(eabuf.at[b2], aggs.at[dstr.at[d4]],
                                  sems[b2]).wait()

        fire_idx_loads(0, 0, 0)
        fire_idx_loads(1, 1, 1)
        wait_idx_loads(0, 0, 0)
        build_and_fire(0, 0)

        @pl.loop(0, n_chunks + 3, step=4)
        def _quad(t0):
            for qq in range(4):
                t = t0 + qq
                b2 = qq % 2
                d4 = qq

                @pl.when(t + 2 < n_chunks)
                def _():
                    fire_idx_loads(t + 2, b2, (d4 + 2) % 4)

                @pl.when((t >= 1) & (t < n_chunks))
                def _():
                    wait_scatter(1 - b2, (d4 + 3) % 4)

                @pl.when(t + 1 < n_chunks)
                def _():
                    wait_idx_loads(t + 1, 1 - b2, (d4 + 1) % 4)
                    build_and_fire(t + 1, 1 - b2)

                @pl.when(t < n_chunks)
                def _():
                    wait_gathers(t, b2)

                    @pl.loop(0, CHUNK)
                    def _rows(r):
                        for j in range(DH // LANES):
                            sl = pl.ds(j * LANES, LANES)
                            eabuf[b2, r, sl] = jnp.maximum(
                                xbuf[b2, r, sl] + eabuf[b2, r, sl], 0.0)

                    pltpu.async_copy(eabuf.at[b2], aggs.at[dstr.at[d4]],
                                     sems[b2], add=True)

        wait_scatter((n_chunks - 1) % 2, (n_chunks - 1) % 4)
        plsc.subcore_barrier()

        @pl.when(s < NSUB - 1)
        def _():
            pltpu.sync_copy(aggs.at[pl.ds(r0, rows_a)],
                            out_h.at[pl.ds(o0, rows_a)])

        @pl.when(s == NSUB - 1)
        def _():
            pltpu.sync_copy(
                aggs.at[pl.ds((NSUB - 1) * rows_a, rows_last)],
                out_h.at[pl.ds(pl.multiple_of(c * N, 8) + (NSUB - 1) * rows_a,
                               rows_last)])

    return agg_kernel


def _mlp_body(eps_ref, x_ref, a0_ref, a1_ref, w1_ref, b1_ref, w2_ref, b2_ref, o_ref):
    agg = jnp.concatenate([a0_ref[...], a1_ref[...]], axis=1)
    h = x_ref[...] * (1.0 + eps_ref[0, 0]) + agg
    h = jnp.maximum(jnp.dot(h, w1_ref[...], preferred_element_type=jnp.float32)
                    + b1_ref[...], 0.0)
    h = jnp.dot(h, w2_ref[...], preferred_element_type=jnp.float32) + b2_ref[...]
    o_ref[...] = jnp.maximum(h, 0.0)


@functools.lru_cache(maxsize=None)
def _make_mlp(N, D, BN):
    DH = D // 2
    nb = N // BN
    assert N % BN == 0
    return pl.pallas_call(
        _mlp_body,
        grid=(nb,),
        in_specs=[
            pl.BlockSpec(memory_space=pltpu.SMEM),
            pl.BlockSpec((BN, D), lambda i: (i, 0)),
            pl.BlockSpec((BN, DH), lambda i: (i, 0)),
            pl.BlockSpec((BN, DH), lambda i: (i + nb, 0)),
            pl.BlockSpec((D, D), lambda i: (0, 0)),
            pl.BlockSpec((1, D), lambda i: (0, 0)),
            pl.BlockSpec((D, D), lambda i: (0, 0)),
            pl.BlockSpec((1, D), lambda i: (0, 0)),
        ],
        out_specs=pl.BlockSpec((BN, D), lambda i: (i, 0)),
        out_shape=jax.ShapeDtypeStruct((N, D), jnp.float32),
    )


def kernel(x, edge_index, edge_attr, W1, b1, W2, b2, eps):
    N, D = x.shape
    E = edge_index.shape[1]
    DH = D // 2
    xh = x.reshape(N * 2, DH)
    src = edge_index[0]
    dst = edge_index[1]
    rows_last = N - (NSUB - 1) * ((N // NSUB) // 8 * 8)
    zeros = jnp.zeros((rows_last, DH), jnp.float32)
    aggflat = _make_agg(N, E, DH)(xh, src, dst, edge_attr, zeros)

    li = W1.shape[0] - 1
    out = _make_mlp(N, D, 1000)(
        eps[li].reshape(1, 1), x, aggflat, aggflat,
        W1[li], b1[li].reshape(1, D), W2[li], b2[li].reshape(1, D))
    return out
